# Initial kernel scaffold; baseline (speedup 1.0000x reference)
import jax
import jax.numpy as jnp
from jax.experimental import pallas as pl


def kernel(spatial_rgb, dist_and_index_list, c):
    raise NotImplementedError("write your pallas kernel here")


# speedup vs baseline: 22.8668x; 22.8668x over previous
import functools
import jax
import jax.numpy as jnp
from jax import lax
from jax.experimental import pallas as pl
from jax.experimental.pallas import tpu as pltpu
from jax.experimental.pallas import tpu_sc as plsc

_N_POINTS = 100000
_B, _H, _W, _K = 4, 384, 384, 8
_M = _B * _H * _W
_HWK = _H * _W * _K
_N_CH = 4
_G = 8
_PPT = _M // _G
_PBLK = 512
_NBLK = _PPT // _PBLK
_NGRP = _PBLK // 16


def _build_sc_kernel():
    mesh = plsc.VectorSubcoreMesh(core_axis_name="c", subcore_axis_name="s")

    @functools.partial(
        pl.kernel,
        out_type=jax.ShapeDtypeStruct((_N_CH * _M,), jnp.float32),
        mesh=mesh,
        scratch_types=[
            pltpu.VMEM((_N_POINTS,), jnp.float32),
            pltpu.VMEM((_PBLK * _K,), jnp.float32),
            pltpu.VMEM((_PBLK * _K,), jnp.float32),
            pltpu.VMEM((_PBLK,), jnp.float32),
            pltpu.VMEM((16,), jnp.float32),
        ],
        compiler_params=pltpu.CompilerParams(needs_layout_passes=False),
    )
    def gauss_sc(table_flat, dii_flat, c16_hbm, out_hbm, col, dbuf, ibuf,
                 obuf, cvm):
        wid = lax.axis_index("s") * 2 + lax.axis_index("c")
        ch = wid % _N_CH
        g = wid // _N_CH
        b = g // 2
        inb0 = (g % 2) * _PPT

        pltpu.sync_copy(table_flat.at[pl.ds(ch * _N_POINTS, _N_POINTS)], col)
        pltpu.sync_copy(c16_hbm, cvm)
        cv = cvm[...]
        scale = -0.5 / (cv * cv)
        lane8 = lax.iota(jnp.int32, 16) * _K
        zero = jnp.zeros((16,), jnp.float32)

        def blk_body(blk, carry):
            d_off = b * (2 * _HWK) + (inb0 + blk * _PBLK) * _K
            pltpu.sync_copy(dii_flat.at[pl.ds(d_off, _PBLK * _K)], dbuf)
            pltpu.sync_copy(dii_flat.at[pl.ds(d_off + _HWK, _PBLK * _K)], ibuf)

            def grp_body(grp, c2):
                base = lane8 + grp * (16 * _K)
                ds = zero
                acc = zero
                for k in range(_K):
                    off = base + k
                    dk = plsc.load_gather(dbuf, [off])
                    w = jnp.exp(dk * dk * scale)
                    ds = ds + w
                    ikf = plsc.load_gather(ibuf, [off])
                    ik = ikf.astype(jnp.int32)
                    xk = plsc.load_gather(col, [ik])
                    acc = acc + w * xk
                r = acc / (ds + 0.001)
                obuf[pl.ds(grp * 16, 16)] = jnp.where(ds > 0, r, 0.0)
                return c2

            lax.fori_loop(0, _NGRP, grp_body, 0)
            out_off = ch * _M + g * _PPT + blk * _PBLK
            pltpu.sync_copy(obuf, out_hbm.at[pl.ds(out_off, _PBLK)])
            return carry

        lax.fori_loop(0, _NBLK, blk_body, 0)

    return gauss_sc


_GAUSS_SC = _build_sc_kernel()


@jax.jit
def kernel(spatial_rgb, dist_and_index_list, c):
    table_flat = spatial_rgb.T.reshape(-1)
    dii_flat = dist_and_index_list.reshape(-1)
    c16 = jnp.broadcast_to(c.reshape(1), (16,)).astype(jnp.float32)
    out = _GAUSS_SC(table_flat, dii_flat, c16)
    return out.reshape(_N_CH, _M).T.reshape(_B, _H, _W, _N_CH)


# speedup vs baseline: 24.6312x; 1.0772x over previous
import functools
import jax
import jax.numpy as jnp
from jax import lax
from jax.experimental import pallas as pl
from jax.experimental.pallas import tpu as pltpu
from jax.experimental.pallas import tpu_sc as plsc

_N_POINTS = 100000
_B, _H, _W, _K = 4, 384, 384, 8
_M = _B * _H * _W
_HWK = _H * _W * _K
_N_CH = 4
_G = 8
_PPT = _M // _G
_PBLK = 768
_NBLK = _PPT // _PBLK
_NGRP = _PBLK // 16


def _build_sc_kernel():
    mesh = plsc.VectorSubcoreMesh(core_axis_name="c", subcore_axis_name="s")

    @functools.partial(
        pl.kernel,
        out_type=jax.ShapeDtypeStruct((_N_CH * _M,), jnp.float32),
        mesh=mesh,
        scratch_types=[
            pltpu.VMEM((_N_POINTS,), jnp.float32),
            pltpu.VMEM((_PBLK, _K), jnp.float32),
            pltpu.VMEM((_PBLK, _K), jnp.float32),
            pltpu.VMEM((_PBLK,), jnp.float32),
            pltpu.VMEM((16,), jnp.float32),
        ],
        compiler_params=pltpu.CompilerParams(
            needs_layout_passes=False, use_tc_tiling_on_sc=False),
    )
    def gauss_sc(table_flat, dii4, c16_hbm, out_hbm, col, dbuf, ibuf,
                 obuf, cvm):
        wid = lax.axis_index("s") * 2 + lax.axis_index("c")
        ch = wid % _N_CH
        g = wid // _N_CH
        b = g // 2
        inb0 = (g % 2) * _PPT

        pltpu.sync_copy(table_flat.at[pl.ds(ch * _N_POINTS, _N_POINTS)], col)
        pltpu.sync_copy(c16_hbm, cvm)
        cv = cvm[...]
        scale = -0.5 / (cv * cv)
        lane = lax.iota(jnp.int32, 16)
        zero = jnp.zeros((16,), jnp.float32)

        def blk_body(blk, carry):
            p0 = inb0 + blk * _PBLK
            pltpu.sync_copy(dii4.at[b, 0, pl.ds(p0, _PBLK), :], dbuf)
            pltpu.sync_copy(dii4.at[b, 1, pl.ds(p0, _PBLK), :], ibuf)

            def grp_body(grp, c2):
                px = lane + grp * 16
                ds = zero
                acc = zero
                for k in range(_K):
                    kv = jnp.full((16,), k, jnp.int32)
                    dk = plsc.load_gather(dbuf, [px, kv])
                    w = jnp.exp(dk * dk * scale)
                    ds = ds + w
                    ikf = plsc.load_gather(ibuf, [px, kv])
                    ik = ikf.astype(jnp.int32)
                    xk = plsc.load_gather(col, [ik])
                    acc = acc + w * xk
                r = acc / (ds + 0.001)
                obuf[pl.ds(grp * 16, 16)] = jnp.where(ds > 0, r, 0.0)
                return c2

            lax.fori_loop(0, _NGRP, grp_body, 0)
            out_off = ch * _M + g * _PPT + blk * _PBLK
            pltpu.sync_copy(obuf, out_hbm.at[pl.ds(out_off, _PBLK)])
            return carry

        lax.fori_loop(0, _NBLK, blk_body, 0)

    return gauss_sc


_GAUSS_SC = _build_sc_kernel()


@jax.jit
def kernel(spatial_rgb, dist_and_index_list, c):
    table_flat = spatial_rgb.T.reshape(-1)
    dii4 = dist_and_index_list.reshape(_B, 2, _H * _W, _K)
    c16 = jnp.broadcast_to(c.reshape(1), (16,)).astype(jnp.float32)
    out = _GAUSS_SC(table_flat, dii4, c16)
    return out.reshape(_N_CH, _M).T.reshape(_B, _H, _W, _N_CH)


# speedup vs baseline: 59.3036x; 2.4077x over previous
import functools
import jax
import jax.numpy as jnp
from jax import lax
from jax.experimental import pallas as pl
from jax.experimental.pallas import tpu as pltpu
from jax.experimental.pallas import tpu_sc as plsc

_N_POINTS = 100000
_B, _H, _W, _K = 4, 384, 384, 8
_M = _B * _H * _W
_HWK = _H * _W * _K
_N_CH = 4
_G = 8
_PPT = _M // _G
_RPB = 2
_PBLK = _RPB * _W
_NBLK = _PPT // _PBLK
_NGRP = _W // 16


def _build_sc_kernel():
    mesh = plsc.VectorSubcoreMesh(core_axis_name="c", subcore_axis_name="s")

    @functools.partial(
        pl.kernel,
        out_type=jax.ShapeDtypeStruct((_N_CH * _M,), jnp.float32),
        mesh=mesh,
        scratch_types=[
            pltpu.VMEM((_N_POINTS,), jnp.float32),
            pltpu.VMEM((_RPB, _K, _W), jnp.float32),
            pltpu.VMEM((_RPB, _K, _W), jnp.float32),
            pltpu.VMEM((_PBLK,), jnp.float32),
            pltpu.VMEM((16,), jnp.float32),
        ],
        compiler_params=pltpu.CompilerParams(
            needs_layout_passes=False, use_tc_tiling_on_sc=False),
    )
    def gauss_sc(table_flat, dii_t, c16_hbm, out_hbm, col, dbuf, ibuf,
                 obuf, cvm):
        wid = lax.axis_index("s") * 2 + lax.axis_index("c")
        ch = wid % _N_CH
        g = wid // _N_CH
        b = g // 2
        h0 = (g % 2) * (_PPT // _W)

        pltpu.sync_copy(table_flat.at[pl.ds(ch * _N_POINTS, _N_POINTS)], col)
        pltpu.sync_copy(c16_hbm, cvm)
        cv = cvm[...]
        scale = -0.5 / (cv * cv)
        zero = jnp.zeros((16,), jnp.float32)

        def blk_body(blk, carry):
            h = h0 + blk * _RPB
            pltpu.sync_copy(dii_t.at[b, 0, pl.ds(h, _RPB), :, :], dbuf)
            pltpu.sync_copy(dii_t.at[b, 1, pl.ds(h, _RPB), :, :], ibuf)

            for r in range(_RPB):
                def grp_body(grp, c2, r=r):
                    w0 = grp * 16
                    ds = zero
                    acc = zero
                    for k in range(_K):
                        dk = dbuf[r, k, pl.ds(w0, 16)]
                        w = jnp.exp(dk * dk * scale)
                        ds = ds + w
                        ik = ibuf[r, k, pl.ds(w0, 16)].astype(jnp.int32)
                        xk = plsc.load_gather(col, [ik])
                        acc = acc + w * xk
                    res = acc / (ds + 0.001)
                    obuf[pl.ds(r * _W + w0, 16)] = jnp.where(ds > 0, res, 0.0)
                    return c2

                lax.fori_loop(0, _NGRP, grp_body, 0)
            out_off = ch * _M + g * _PPT + blk * _PBLK
            pltpu.sync_copy(obuf, out_hbm.at[pl.ds(out_off, _PBLK)])
            return carry

        lax.fori_loop(0, _NBLK, blk_body, 0)

    return gauss_sc


_GAUSS_SC = _build_sc_kernel()


@jax.jit
def kernel(spatial_rgb, dist_and_index_list, c):
    table_flat = spatial_rgb.T.reshape(-1)
    dii_t = jnp.transpose(dist_and_index_list, (0, 1, 2, 4, 3))
    c16 = jnp.broadcast_to(c.reshape(1), (16,)).astype(jnp.float32)
    out = _GAUSS_SC(table_flat, dii_t, c16)
    return out.reshape(_N_CH, _M).T.reshape(_B, _H, _W, _N_CH)


# speedup vs baseline: 113.7059x; 1.9174x over previous
import functools
import jax
import jax.numpy as jnp
from jax import lax
from jax.experimental import pallas as pl
from jax.experimental.pallas import tpu as pltpu
from jax.experimental.pallas import tpu_sc as plsc

_N_POINTS = 100000
_B, _H, _W, _K = 4, 384, 384, 8
_M = _B * _H * _W
_HWK = _H * _W * _K
_N_CH = 4
_G = 8
_PPT = _M // _G
_RPB = 2
_PBLK = _RPB * _W
_NBLK = _PPT // _PBLK
_NGRP = _W // 16


def _build_sc_kernel():
    mesh = plsc.VectorSubcoreMesh(core_axis_name="c", subcore_axis_name="s")

    @functools.partial(
        pl.kernel,
        out_type=jax.ShapeDtypeStruct((_N_CH * _M,), jnp.float32),
        mesh=mesh,
        scratch_types=[
            pltpu.VMEM((_N_POINTS,), jnp.float32),
            pltpu.VMEM((_RPB, _K, _W), jnp.float32),
            pltpu.VMEM((_RPB, _K, _W), jnp.float32),
            pltpu.VMEM((_RPB, _K, _W), jnp.float32),
            pltpu.VMEM((_RPB, _K, _W), jnp.float32),
            pltpu.VMEM((_PBLK,), jnp.float32),
            pltpu.VMEM((_PBLK,), jnp.float32),
            pltpu.VMEM((16,), jnp.float32),
            pltpu.SemaphoreType.DMA,
            pltpu.SemaphoreType.DMA,
            pltpu.SemaphoreType.DMA,
            pltpu.SemaphoreType.DMA,
            pltpu.SemaphoreType.DMA,
            pltpu.SemaphoreType.DMA,
        ],
        compiler_params=pltpu.CompilerParams(
            needs_layout_passes=False, use_tc_tiling_on_sc=False),
    )
    def gauss_sc(table_flat, dii_t, c16_hbm, out_hbm, col, dbuf0, dbuf1,
                 ibuf0, ibuf1, obuf0, obuf1, cvm, sd0, sd1, si0, si1,
                 so0, so1):
        wid = lax.axis_index("s") * 2 + lax.axis_index("c")
        ch = wid % _N_CH
        g = wid // _N_CH
        b = g // 2
        h0 = (g % 2) * (_PPT // _W)

        dbufs, ibufs, obufs = (dbuf0, dbuf1), (ibuf0, ibuf1), (obuf0, obuf1)
        sds, sis, sos = (sd0, sd1), (si0, si1), (so0, so1)

        def d_src(blk):
            return dii_t.at[b, 0, pl.ds(h0 + blk * _RPB, _RPB), :, :]

        def i_src(blk):
            return dii_t.at[b, 1, pl.ds(h0 + blk * _RPB, _RPB), :, :]

        def o_dst(blk):
            return out_hbm.at[pl.ds(ch * _M + g * _PPT + blk * _PBLK, _PBLK)]

        pltpu.sync_copy(table_flat.at[pl.ds(ch * _N_POINTS, _N_POINTS)], col)
        pltpu.sync_copy(c16_hbm, cvm)
        cv = cvm[...]
        scale = -0.5 / (cv * cv)
        zero = jnp.zeros((16,), jnp.float32)

        def compute(dbuf, ibuf, obuf):
            for r in range(_RPB):
                def grp_body(grp, c2, r=r):
                    w0 = grp * 16
                    ds = zero
                    acc = zero
                    for k in range(_K):
                        dk = dbuf[r, k, pl.ds(w0, 16)]
                        w = jnp.exp(dk * dk * scale)
                        ds = ds + w
                        ik = ibuf[r, k, pl.ds(w0, 16)].astype(jnp.int32)
                        xk = plsc.load_gather(col, [ik])
                        acc = acc + w * xk
                    res = acc / (ds + 0.001)
                    obuf[pl.ds(r * _W + w0, 16)] = jnp.where(ds > 0, res, 0.0)
                    return c2

                lax.fori_loop(0, _NGRP, grp_body, 0)

        for s in range(2):
            pltpu.async_copy(d_src(s), dbufs[s], sds[s])
            pltpu.async_copy(i_src(s), ibufs[s], sis[s])

        def outer(i, carry):
            for s in range(2):
                blk = i * 2 + s
                pltpu.make_async_copy(d_src(blk), dbufs[s], sds[s]).wait()
                pltpu.make_async_copy(i_src(blk), ibufs[s], sis[s]).wait()

                @pl.when(blk >= 2)
                def _():
                    pltpu.make_async_copy(obufs[s], o_dst(blk - 2),
                                          sos[s]).wait()

                compute(dbufs[s], ibufs[s], obufs[s])
                pltpu.async_copy(obufs[s], o_dst(blk), sos[s])

                @pl.when(blk + 2 < _NBLK)
                def _():
                    pltpu.async_copy(d_src(blk + 2), dbufs[s], sds[s])
                    pltpu.async_copy(i_src(blk + 2), ibufs[s], sis[s])
            return carry

        lax.fori_loop(0, _NBLK // 2, outer, 0)
        pltpu.make_async_copy(obuf0, o_dst(_NBLK - 2), so0).wait()
        pltpu.make_async_copy(obuf1, o_dst(_NBLK - 1), so1).wait()

    return gauss_sc


_GAUSS_SC = _build_sc_kernel()


@jax.jit
def kernel(spatial_rgb, dist_and_index_list, c):
    table_flat = spatial_rgb.T.reshape(-1)
    dii_t = jnp.transpose(dist_and_index_list, (0, 1, 2, 4, 3))
    c16 = jnp.broadcast_to(c.reshape(1), (16,)).astype(jnp.float32)
    out = _GAUSS_SC(table_flat, dii_t, c16)
    return out.reshape(_N_CH, _M).T.reshape(_B, _H, _W, _N_CH)
